# Initial kernel scaffold; baseline (speedup 1.0000x reference)
import jax
import jax.numpy as jnp
from jax.experimental import pallas as pl


def kernel(x, edge_index, W1, b1, W2, b2, W3, b3, Wp1, bp1, Wp2, bp2, Wv1, bv1, Wv2, bv2):
    raise NotImplementedError("write your pallas kernel here")


# speedup vs baseline: 7.6203x; 7.6203x over previous
import functools

import jax
import jax.numpy as jnp
from jax import lax
from jax.experimental import pallas as pl
from jax.experimental.pallas import tpu as pltpu
from jax.experimental.pallas import tpu_sc as plsc

N = 10000
D = 128
A = 6
E = 320000
NC, NS = 2, 16
NW = NC * NS
CH = 128
CHUNKS = 80
EPW = CH * CHUNKS
E_PAD = EPW * NW
DUMP = N
N_ACC = N + 112
RPT = N_ACC // NS
DEG_W = 16
BLK = 400
GRID = N // BLK

_MESH = plsc.VectorSubcoreMesh(core_axis_name="c", subcore_axis_name="s")


@functools.partial(
    pl.kernel,
    out_type=jax.ShapeDtypeStruct((NC, N_ACC, DEG_W), jnp.float32),
    mesh=_MESH,
    scratch_types=[
        pltpu.VMEM((CHUNKS, CH), jnp.int32),
        pltpu.VMEM((CH, DEG_W), jnp.float32),
        pltpu.VMEM_SHARED((N_ACC, DEG_W), jnp.float32),
    ],
    compiler_params=pltpu.CompilerParams(use_tc_tiling_on_sc=False),
)
def _sc_degree(dst_hbm, ones_hbm, zz_hbm, out_hbm, dst_v, ones_v, acc_sh):
    c = lax.axis_index("c")
    s = lax.axis_index("s")
    wid = s * NC + c
    pltpu.sync_copy(dst_hbm.at[wid], dst_v)
    pltpu.sync_copy(ones_hbm, ones_v)
    pltpu.sync_copy(zz_hbm, acc_sh.at[pl.ds(s * RPT, RPT)])
    plsc.subcore_barrier()

    def step(j, carry):
        pltpu.sync_copy(ones_v, acc_sh.at[dst_v.at[j]], add=True)
        return carry

    lax.fori_loop(0, CHUNKS, step, 0, unroll=False)
    plsc.subcore_barrier()
    pltpu.sync_copy(acc_sh.at[pl.ds(s * RPT, RPT)],
                    out_hbm.at[c, pl.ds(s * RPT, RPT)])


@functools.partial(
    pl.kernel,
    out_type=jax.ShapeDtypeStruct((NC, N_ACC, D), jnp.float32),
    mesh=_MESH,
    scratch_types=[
        pltpu.VMEM((CHUNKS, CH), jnp.int32),
        pltpu.VMEM((CHUNKS, CH), jnp.int32),
        pltpu.VMEM((CH, D), jnp.float32),
        pltpu.SemaphoreType.DMA,
        pltpu.VMEM_SHARED((N_ACC, D), jnp.float32),
    ],
)
def _sc_propagate(hp_hbm, src_hbm, dst_hbm, zz_hbm, out_hbm,
                  src_v, dst_v, rows_v, sem, acc_sh):
    c = lax.axis_index("c")
    s = lax.axis_index("s")
    wid = s * NC + c
    pltpu.sync_copy(src_hbm.at[wid], src_v)
    pltpu.sync_copy(dst_hbm.at[wid], dst_v)
    pltpu.sync_copy(zz_hbm, acc_sh.at[pl.ds(s * RPT, RPT)])
    plsc.subcore_barrier()

    def step(j, carry):
        pltpu.async_copy(hp_hbm.at[src_v.at[j]], rows_v, sem).wait()
        pltpu.sync_copy(rows_v, acc_sh.at[dst_v.at[j]], add=True)
        return carry

    lax.fori_loop(0, CHUNKS, step, 0, unroll=False)
    plsc.subcore_barrier()
    pltpu.sync_copy(acc_sh.at[pl.ds(s * RPT, RPT)],
                    out_hbm.at[c, pl.ds(s * RPT, RPT)])


def _tc_in_body(x_ref, w_ref, d0_ref, d1_ref, hp_ref, dinv_ref):
    deg = d0_ref[0, :, :1] + d1_ref[0, :, :1] + 1.0
    dinv = lax.rsqrt(deg)
    h = jnp.dot(x_ref[...], w_ref[...], preferred_element_type=jnp.float32,
                   precision=lax.Precision.HIGHEST)
    hp_ref[...] = h * dinv
    dinv_ref[...] = dinv


_tc_input = pl.pallas_call(
    _tc_in_body,
    grid=(GRID,),
    in_specs=[
        pl.BlockSpec((BLK, D), lambda i: (i, 0)),
        pl.BlockSpec((D, D), lambda i: (0, 0)),
        pl.BlockSpec((1, BLK, DEG_W), lambda i: (0, i, 0)),
        pl.BlockSpec((1, BLK, DEG_W), lambda i: (1, i, 0)),
    ],
    out_specs=[
        pl.BlockSpec((BLK, D), lambda i: (i, 0)),
        pl.BlockSpec((BLK, 1), lambda i: (i, 0)),
    ],
    out_shape=[
        jax.ShapeDtypeStruct((N, D), jnp.float32),
        jax.ShapeDtypeStruct((N, 1), jnp.float32),
    ],
)


def _tc_mid_body(p_ref, hp_ref, dinv_ref, b_ref, w_ref, out_ref):
    s = p_ref[0] + p_ref[1] + hp_ref[...]
    dinv = dinv_ref[...]
    t = jnp.maximum(s * dinv + b_ref[...], 0.0)
    out_ref[...] = jnp.dot(t, w_ref[...],
                           preferred_element_type=jnp.float32,
                   precision=lax.Precision.HIGHEST) * dinv


_tc_mid = pl.pallas_call(
    _tc_mid_body,
    grid=(GRID,),
    in_specs=[
        pl.BlockSpec((NC, BLK, D), lambda i: (0, i, 0)),
        pl.BlockSpec((BLK, D), lambda i: (i, 0)),
        pl.BlockSpec((BLK, 1), lambda i: (i, 0)),
        pl.BlockSpec((1, D), lambda i: (0, 0)),
        pl.BlockSpec((D, D), lambda i: (0, 0)),
    ],
    out_specs=pl.BlockSpec((BLK, D), lambda i: (i, 0)),
    out_shape=jax.ShapeDtypeStruct((N, D), jnp.float32),
)


def _tc_head_body(p_ref, hp_ref, dinv_ref, b3_ref,
                  wp1_ref, bp1_ref, wp2_ref, bp2_ref,
                  wv1_ref, bv1_ref, wv2_ref, bv2_ref,
                  lo_ref, vo_ref, acc_ref):
    i = pl.program_id(0)
    s = p_ref[0] + p_ref[1] + hp_ref[...]
    t = s * dinv_ref[...] + b3_ref[...]
    csum = jnp.sum(t, axis=0, keepdims=True)

    @pl.when(i == 0)
    def _():
        acc_ref[...] = csum

    @pl.when(i > 0)
    def _():
        acc_ref[...] += csum

    @pl.when(i == GRID - 1)
    def _():
        g = acc_ref[...] * (1.0 / N)
        hp_pol = jnp.maximum(
            jnp.dot(g, wp1_ref[...], preferred_element_type=jnp.float32,
                   precision=lax.Precision.HIGHEST)
            + bp1_ref[...], 0.0)
        lo_ref[...] = jnp.dot(hp_pol, wp2_ref[...],
                              preferred_element_type=jnp.float32,
                   precision=lax.Precision.HIGHEST) + bp2_ref[...]
        hp_val = jnp.maximum(
            jnp.dot(g, wv1_ref[...], preferred_element_type=jnp.float32,
                   precision=lax.Precision.HIGHEST)
            + bv1_ref[...], 0.0)
        vo_ref[...] = jnp.dot(hp_val, wv2_ref[...],
                              preferred_element_type=jnp.float32,
                   precision=lax.Precision.HIGHEST) + bv2_ref[...]


_tc_head = pl.pallas_call(
    _tc_head_body,
    grid=(GRID,),
    in_specs=[
        pl.BlockSpec((NC, BLK, D), lambda i: (0, i, 0)),
        pl.BlockSpec((BLK, D), lambda i: (i, 0)),
        pl.BlockSpec((BLK, 1), lambda i: (i, 0)),
        pl.BlockSpec((1, D), lambda i: (0, 0)),
        pl.BlockSpec((D, D), lambda i: (0, 0)),
        pl.BlockSpec((1, D), lambda i: (0, 0)),
        pl.BlockSpec((D, D), lambda i: (0, 0)),
        pl.BlockSpec((1, D), lambda i: (0, 0)),
        pl.BlockSpec((D, D), lambda i: (0, 0)),
        pl.BlockSpec((1, D), lambda i: (0, 0)),
        pl.BlockSpec((D, D), lambda i: (0, 0)),
        pl.BlockSpec((1, D), lambda i: (0, 0)),
    ],
    out_specs=[
        pl.BlockSpec((1, D), lambda i: (0, 0)),
        pl.BlockSpec((1, D), lambda i: (0, 0)),
    ],
    out_shape=[
        jax.ShapeDtypeStruct((1, D), jnp.float32),
        jax.ShapeDtypeStruct((1, D), jnp.float32),
    ],
    scratch_shapes=[pltpu.VMEM((1, D), jnp.float32)],
)


def kernel(x, edge_index, W1, b1, W2, b2, W3, b3,
           Wp1, bp1, Wp2, bp2, Wv1, bv1, Wv2, bv2):
    src = edge_index[0]
    dst = edge_index[1]
    pad = E_PAD - E
    srcp = jnp.concatenate(
        [src, jnp.zeros((pad,), jnp.int32)]).reshape(NW, CHUNKS, CH)
    dstp = jnp.concatenate(
        [dst, jnp.full((pad,), DUMP, jnp.int32)]).reshape(NW, CHUNKS, CH)
    zz = jnp.zeros((RPT, D), jnp.float32)
    zz16 = jnp.zeros((RPT, DEG_W), jnp.float32)
    ones16 = jnp.ones((CH, DEG_W), jnp.float32)

    degp = _sc_degree(dstp, ones16, zz16)
    hp1, dinv = _tc_input(x, W1, degp, degp)
    p1 = _sc_propagate(hp1, srcp, dstp, zz)
    hp2 = _tc_mid(p1, hp1, dinv, b1.reshape(1, D), W2)
    p2 = _sc_propagate(hp2, srcp, dstp, zz)
    hp3 = _tc_mid(p2, hp2, dinv, b2.reshape(1, D), W3)
    p3 = _sc_propagate(hp3, srcp, dstp, zz)

    wp2p = jnp.zeros((D, D), jnp.float32).at[:, :A].set(Wp2)
    bp2p = jnp.zeros((1, D), jnp.float32).at[0, :A].set(bp2)
    wv2p = jnp.zeros((D, D), jnp.float32).at[:, :1].set(Wv2)
    bv2p = jnp.zeros((1, D), jnp.float32).at[0, :1].set(bv2)
    logits, value = _tc_head(
        p3, hp3, dinv, b3.reshape(1, D),
        Wp1, bp1.reshape(1, D), wp2p, bp2p,
        Wv1, bv1.reshape(1, D), wv2p, bv2p)
    return logits[:, :A], value[:, :1]


# speedup vs baseline: 8.7209x; 1.1444x over previous
import functools

import jax
import jax.numpy as jnp
from jax import lax
from jax.experimental import pallas as pl
from jax.experimental.pallas import tpu as pltpu
from jax.experimental.pallas import tpu_sc as plsc

N = 10000
D = 128
A = 6
E = 320000
NC, NS = 2, 16
NW = NC * NS
CH = 80
CHUNKS = 128
EPW = CH * CHUNKS
E_PAD = EPW * NW
DUMP = N
N_ACC = N + 112
RPT = N_ACC // NS
DEG_W = 16
BLK = 400
GRID = N // BLK

_MESH = plsc.VectorSubcoreMesh(core_axis_name="c", subcore_axis_name="s")


@functools.partial(
    pl.kernel,
    out_type=jax.ShapeDtypeStruct((NC, N_ACC, DEG_W), jnp.float32),
    mesh=_MESH,
    scratch_types=[
        pltpu.VMEM((CHUNKS, CH), jnp.int32),
        pltpu.VMEM((CH, DEG_W), jnp.float32),
        pltpu.VMEM_SHARED((N_ACC, DEG_W), jnp.float32),
    ],
    compiler_params=pltpu.CompilerParams(use_tc_tiling_on_sc=False),
)
def _sc_degree(dst_hbm, ones_hbm, zz_hbm, out_hbm, dst_v, ones_v, acc_sh):
    c = lax.axis_index("c")
    s = lax.axis_index("s")
    wid = s * NC + c
    pltpu.sync_copy(dst_hbm.at[wid], dst_v)
    pltpu.sync_copy(ones_hbm, ones_v)
    pltpu.sync_copy(zz_hbm, acc_sh.at[pl.ds(s * RPT, RPT)])
    plsc.subcore_barrier()

    def step(j, carry):
        pltpu.sync_copy(ones_v, acc_sh.at[dst_v.at[j]], add=True)
        return carry

    lax.fori_loop(0, CHUNKS, step, 0, unroll=False)
    plsc.subcore_barrier()
    pltpu.sync_copy(acc_sh.at[pl.ds(s * RPT, RPT)],
                    out_hbm.at[c, pl.ds(s * RPT, RPT)])


NBUF = 2


@functools.partial(
    pl.kernel,
    out_type=jax.ShapeDtypeStruct((NC, N_ACC, D), jnp.float32),
    mesh=_MESH,
    scratch_types=[
        pltpu.VMEM((CHUNKS, CH), jnp.int32),
        pltpu.VMEM((CHUNKS, CH), jnp.int32),
        pltpu.VMEM((NBUF, CH, D), jnp.float32),
    ] + [pltpu.SemaphoreType.DMA] * NBUF + [
        pltpu.VMEM_SHARED((N_ACC, D), jnp.float32),
    ],
    compiler_params=pltpu.CompilerParams(use_tc_tiling_on_sc=False),
)
def _sc_propagate(hp_hbm, src_hbm, dst_hbm, zz_hbm, out_hbm,
                  src_v, dst_v, rows_v, g0, g1, acc_sh):
    gsems = (g0, g1)
    c = lax.axis_index("c")
    s = lax.axis_index("s")
    wid = s * NC + c
    pltpu.sync_copy(src_hbm.at[wid], src_v)
    pltpu.sync_copy(dst_hbm.at[wid], dst_v)
    pltpu.sync_copy(zz_hbm, acc_sh.at[pl.ds(s * RPT, RPT)])
    plsc.subcore_barrier()

    for b in range(NBUF):
        pltpu.async_copy(hp_hbm.at[src_v.at[b]], rows_v.at[b], gsems[b])

    def group(g, carry):
        for b in range(NBUF):
            j = g * NBUF + b
            pltpu.make_async_copy(hp_hbm.at[src_v.at[b]],
                                  rows_v.at[b], gsems[b]).wait()
            pltpu.sync_copy(rows_v.at[b], acc_sh.at[dst_v.at[j]], add=True)

            @pl.when(j + NBUF < CHUNKS)
            def _():
                pltpu.async_copy(hp_hbm.at[src_v.at[j + NBUF]],
                                 rows_v.at[b], gsems[b])
        return carry

    lax.fori_loop(0, CHUNKS // NBUF, group, 0, unroll=False)
    plsc.subcore_barrier()
    pltpu.sync_copy(acc_sh.at[pl.ds(s * RPT, RPT)],
                    out_hbm.at[c, pl.ds(s * RPT, RPT)])


def _tc_in_body(x_ref, w_ref, d0_ref, d1_ref, hp_ref, dinv_ref):
    deg = d0_ref[0, :, :1] + d1_ref[0, :, :1] + 1.0
    dinv = lax.rsqrt(deg)
    h = jnp.dot(x_ref[...], w_ref[...], preferred_element_type=jnp.float32,
                   precision=lax.Precision.HIGHEST)
    hp_ref[...] = h * dinv
    dinv_ref[...] = dinv


_tc_input = pl.pallas_call(
    _tc_in_body,
    grid=(GRID,),
    in_specs=[
        pl.BlockSpec((BLK, D), lambda i: (i, 0)),
        pl.BlockSpec((D, D), lambda i: (0, 0)),
        pl.BlockSpec((1, BLK, DEG_W), lambda i: (0, i, 0)),
        pl.BlockSpec((1, BLK, DEG_W), lambda i: (1, i, 0)),
    ],
    out_specs=[
        pl.BlockSpec((BLK, D), lambda i: (i, 0)),
        pl.BlockSpec((BLK, 1), lambda i: (i, 0)),
    ],
    out_shape=[
        jax.ShapeDtypeStruct((N, D), jnp.float32),
        jax.ShapeDtypeStruct((N, 1), jnp.float32),
    ],
)


def _tc_mid_body(p_ref, hp_ref, dinv_ref, b_ref, w_ref, out_ref):
    s = p_ref[0] + p_ref[1] + hp_ref[...]
    dinv = dinv_ref[...]
    t = jnp.maximum(s * dinv + b_ref[...], 0.0)
    out_ref[...] = jnp.dot(t, w_ref[...],
                           preferred_element_type=jnp.float32,
                   precision=lax.Precision.HIGHEST) * dinv


_tc_mid = pl.pallas_call(
    _tc_mid_body,
    grid=(GRID,),
    in_specs=[
        pl.BlockSpec((NC, BLK, D), lambda i: (0, i, 0)),
        pl.BlockSpec((BLK, D), lambda i: (i, 0)),
        pl.BlockSpec((BLK, 1), lambda i: (i, 0)),
        pl.BlockSpec((1, D), lambda i: (0, 0)),
        pl.BlockSpec((D, D), lambda i: (0, 0)),
    ],
    out_specs=pl.BlockSpec((BLK, D), lambda i: (i, 0)),
    out_shape=jax.ShapeDtypeStruct((N, D), jnp.float32),
)


def _tc_head_body(p_ref, hp_ref, dinv_ref, b3_ref,
                  wp1_ref, bp1_ref, wp2_ref, bp2_ref,
                  wv1_ref, bv1_ref, wv2_ref, bv2_ref,
                  lo_ref, vo_ref, acc_ref):
    i = pl.program_id(0)
    s = p_ref[0] + p_ref[1] + hp_ref[...]
    t = s * dinv_ref[...] + b3_ref[...]
    csum = jnp.sum(t, axis=0, keepdims=True)

    @pl.when(i == 0)
    def _():
        acc_ref[...] = csum

    @pl.when(i > 0)
    def _():
        acc_ref[...] += csum

    @pl.when(i == GRID - 1)
    def _():
        g = acc_ref[...] * (1.0 / N)
        hp_pol = jnp.maximum(
            jnp.dot(g, wp1_ref[...], preferred_element_type=jnp.float32,
                   precision=lax.Precision.HIGHEST)
            + bp1_ref[...], 0.0)
        lo_ref[...] = jnp.dot(hp_pol, wp2_ref[...],
                              preferred_element_type=jnp.float32,
                   precision=lax.Precision.HIGHEST) + bp2_ref[...]
        hp_val = jnp.maximum(
            jnp.dot(g, wv1_ref[...], preferred_element_type=jnp.float32,
                   precision=lax.Precision.HIGHEST)
            + bv1_ref[...], 0.0)
        vo_ref[...] = jnp.dot(hp_val, wv2_ref[...],
                              preferred_element_type=jnp.float32,
                   precision=lax.Precision.HIGHEST) + bv2_ref[...]


_tc_head = pl.pallas_call(
    _tc_head_body,
    grid=(GRID,),
    in_specs=[
        pl.BlockSpec((NC, BLK, D), lambda i: (0, i, 0)),
        pl.BlockSpec((BLK, D), lambda i: (i, 0)),
        pl.BlockSpec((BLK, 1), lambda i: (i, 0)),
        pl.BlockSpec((1, D), lambda i: (0, 0)),
        pl.BlockSpec((D, D), lambda i: (0, 0)),
        pl.BlockSpec((1, D), lambda i: (0, 0)),
        pl.BlockSpec((D, D), lambda i: (0, 0)),
        pl.BlockSpec((1, D), lambda i: (0, 0)),
        pl.BlockSpec((D, D), lambda i: (0, 0)),
        pl.BlockSpec((1, D), lambda i: (0, 0)),
        pl.BlockSpec((D, D), lambda i: (0, 0)),
        pl.BlockSpec((1, D), lambda i: (0, 0)),
    ],
    out_specs=[
        pl.BlockSpec((1, D), lambda i: (0, 0)),
        pl.BlockSpec((1, D), lambda i: (0, 0)),
    ],
    out_shape=[
        jax.ShapeDtypeStruct((1, D), jnp.float32),
        jax.ShapeDtypeStruct((1, D), jnp.float32),
    ],
    scratch_shapes=[pltpu.VMEM((1, D), jnp.float32)],
)


def kernel(x, edge_index, W1, b1, W2, b2, W3, b3,
           Wp1, bp1, Wp2, bp2, Wv1, bv1, Wv2, bv2):
    src = edge_index[0]
    dst = edge_index[1]
    pad = E_PAD - E
    srcp = jnp.concatenate(
        [src, jnp.zeros((pad,), jnp.int32)]).reshape(NW, CHUNKS, CH)
    dstp = jnp.concatenate(
        [dst, jnp.full((pad,), DUMP, jnp.int32)]).reshape(NW, CHUNKS, CH)
    zz = jnp.zeros((RPT, D), jnp.float32)
    zz16 = jnp.zeros((RPT, DEG_W), jnp.float32)
    ones16 = jnp.ones((CH, DEG_W), jnp.float32)

    degp = _sc_degree(dstp, ones16, zz16)
    hp1, dinv = _tc_input(x, W1, degp, degp)
    p1 = _sc_propagate(hp1, srcp, dstp, zz)
    hp2 = _tc_mid(p1, hp1, dinv, b1.reshape(1, D), W2)
    p2 = _sc_propagate(hp2, srcp, dstp, zz)
    hp3 = _tc_mid(p2, hp2, dinv, b2.reshape(1, D), W3)
    p3 = _sc_propagate(hp3, srcp, dstp, zz)

    wp2p = jnp.zeros((D, D), jnp.float32).at[:, :A].set(Wp2)
    bp2p = jnp.zeros((1, D), jnp.float32).at[0, :A].set(bp2)
    wv2p = jnp.zeros((D, D), jnp.float32).at[:, :1].set(Wv2)
    bv2p = jnp.zeros((1, D), jnp.float32).at[0, :1].set(bv2)
    logits, value = _tc_head(
        p3, hp3, dinv, b3.reshape(1, D),
        Wp1, bp1.reshape(1, D), wp2p, bp2p,
        Wv1, bv1.reshape(1, D), wv2p, bv2p)
    return logits[:, :A], value[:, :1]


# speedup vs baseline: 11.8375x; 1.3574x over previous
import functools

import jax
import jax.numpy as jnp
from jax import lax
from jax.experimental import pallas as pl
from jax.experimental.pallas import tpu as pltpu
from jax.experimental.pallas import tpu_sc as plsc

N = 10000
D = 128
A = 6
E = 320000
NC, NS = 2, 16
NW = NC * NS
CH = 128
CHUNKS = 80
EPW = CH * CHUNKS
E_PAD = EPW * NW
DUMP = N
N_ACC = N + 112
RPT = N_ACC // NS
DEG_W = 16
BLK = 400
GRID = N // BLK

_MESH = plsc.VectorSubcoreMesh(core_axis_name="c", subcore_axis_name="s")


@functools.partial(
    pl.kernel,
    out_type=jax.ShapeDtypeStruct((NC, N_ACC, DEG_W), jnp.float32),
    mesh=_MESH,
    scratch_types=[
        pltpu.VMEM((CHUNKS, CH), jnp.int32),
        pltpu.VMEM((CH, DEG_W), jnp.float32),
        pltpu.VMEM_SHARED((N_ACC, DEG_W), jnp.float32),
    ],
    compiler_params=pltpu.CompilerParams(use_tc_tiling_on_sc=False),
)
def _sc_degree(dst_hbm, ones_hbm, zz_hbm, out_hbm, dst_v, ones_v, acc_sh):
    c = lax.axis_index("c")
    s = lax.axis_index("s")
    wid = s * NC + c
    pltpu.sync_copy(dst_hbm.at[wid], dst_v)
    pltpu.sync_copy(ones_hbm, ones_v)
    pltpu.sync_copy(zz_hbm, acc_sh.at[pl.ds(s * RPT, RPT)])
    plsc.subcore_barrier()

    def step(j, carry):
        pltpu.sync_copy(ones_v, acc_sh.at[dst_v.at[j]], add=True)
        return carry

    lax.fori_loop(0, CHUNKS, step, 0, unroll=False)
    plsc.subcore_barrier()
    pltpu.sync_copy(acc_sh.at[pl.ds(s * RPT, RPT)],
                    out_hbm.at[c, pl.ds(s * RPT, RPT)])


NBUF = 4
PRE = 2


@functools.partial(
    pl.kernel,
    out_type=jax.ShapeDtypeStruct((NC, N_ACC, D), jnp.bfloat16),
    mesh=_MESH,
    scratch_types=[
        pltpu.VMEM((CHUNKS, CH), jnp.int32),
        pltpu.VMEM((CHUNKS, CH), jnp.int32),
        pltpu.VMEM((NBUF, CH, D), jnp.bfloat16),
    ] + [pltpu.SemaphoreType.DMA] * (2 * NBUF) + [
        pltpu.VMEM_SHARED((N_ACC, D), jnp.bfloat16),
    ],
    compiler_params=pltpu.CompilerParams(use_tc_tiling_on_sc=False),
)
def _sc_propagate(hp_hbm, src_hbm, dst_hbm, zz_hbm, out_hbm,
                  src_v, dst_v, rows_v,
                  g0, g1, g2, g3, s0, s1, s2, s3, acc_sh):
    gsems = (g0, g1, g2, g3)
    ssems = (s0, s1, s2, s3)
    c = lax.axis_index("c")
    s = lax.axis_index("s")
    wid = s * NC + c
    pltpu.sync_copy(src_hbm.at[wid], src_v)
    pltpu.sync_copy(dst_hbm.at[wid], dst_v)
    pltpu.sync_copy(zz_hbm, acc_sh.at[pl.ds(s * RPT, RPT)])
    plsc.subcore_barrier()

    def gather(j, b):
        pltpu.async_copy(hp_hbm.at[src_v.at[j]], rows_v.at[b], gsems[b])

    def wait_gather(b):
        pltpu.make_async_copy(hp_hbm.at[src_v.at[0]],
                              rows_v.at[b], gsems[b]).wait()

    def scatter(j, b):
        pltpu.async_copy(rows_v.at[b], acc_sh.at[dst_v.at[j]], ssems[b],
                         add=True)

    def wait_scatter(b):
        pltpu.make_async_copy(rows_v.at[b], acc_sh.at[dst_v.at[0]],
                              ssems[b]).wait()

    for b in range(PRE):
        gather(b, b)

    def group(g, carry):
        for b in range(NBUF):
            j = g * NBUF + b
            wait_gather(b)
            scatter(j, b)
            bg = (b + PRE) % NBUF

            @pl.when(j >= NBUF - PRE)
            def _():
                wait_scatter(bg)

            @pl.when(j + PRE < CHUNKS)
            def _():
                gather(j + PRE, bg)
        return carry

    lax.fori_loop(0, CHUNKS // NBUF, group, 0, unroll=False)
    for j in range(CHUNKS - NBUF + PRE, CHUNKS):
        wait_scatter(j % NBUF)
    plsc.subcore_barrier()
    pltpu.sync_copy(acc_sh.at[pl.ds(s * RPT, RPT)],
                    out_hbm.at[c, pl.ds(s * RPT, RPT)])


def _tc_in_body(x_ref, w_ref, d0_ref, d1_ref, hp_ref, hb_ref, dinv_ref):
    deg = d0_ref[0, :, :1] + d1_ref[0, :, :1] + 1.0
    dinv = lax.rsqrt(deg)
    h = jnp.dot(x_ref[...], w_ref[...], preferred_element_type=jnp.float32,
                   precision=lax.Precision.HIGHEST)
    hp = h * dinv
    hp_ref[...] = hp
    hb_ref[...] = hp.astype(jnp.bfloat16)
    dinv_ref[...] = dinv


_tc_input = pl.pallas_call(
    _tc_in_body,
    grid=(GRID,),
    in_specs=[
        pl.BlockSpec((BLK, D), lambda i: (i, 0)),
        pl.BlockSpec((D, D), lambda i: (0, 0)),
        pl.BlockSpec((1, BLK, DEG_W), lambda i: (0, i, 0)),
        pl.BlockSpec((1, BLK, DEG_W), lambda i: (1, i, 0)),
    ],
    out_specs=[
        pl.BlockSpec((BLK, D), lambda i: (i, 0)),
        pl.BlockSpec((BLK, D), lambda i: (i, 0)),
        pl.BlockSpec((BLK, 1), lambda i: (i, 0)),
    ],
    out_shape=[
        jax.ShapeDtypeStruct((N, D), jnp.float32),
        jax.ShapeDtypeStruct((N, D), jnp.bfloat16),
        jax.ShapeDtypeStruct((N, 1), jnp.float32),
    ],
)


def _tc_mid_body(p_ref, hp_ref, dinv_ref, b_ref, w_ref, out_ref, ob_ref):
    s = (p_ref[0].astype(jnp.float32) + p_ref[1].astype(jnp.float32)
         + hp_ref[...])
    dinv = dinv_ref[...]
    t = jnp.maximum(s * dinv + b_ref[...], 0.0)
    o = jnp.dot(t, w_ref[...],
                preferred_element_type=jnp.float32,
                precision=lax.Precision.HIGHEST) * dinv
    out_ref[...] = o
    ob_ref[...] = o.astype(jnp.bfloat16)


_tc_mid = pl.pallas_call(
    _tc_mid_body,
    grid=(GRID,),
    in_specs=[
        pl.BlockSpec((NC, BLK, D), lambda i: (0, i, 0)),
        pl.BlockSpec((BLK, D), lambda i: (i, 0)),
        pl.BlockSpec((BLK, 1), lambda i: (i, 0)),
        pl.BlockSpec((1, D), lambda i: (0, 0)),
        pl.BlockSpec((D, D), lambda i: (0, 0)),
    ],
    out_specs=[
        pl.BlockSpec((BLK, D), lambda i: (i, 0)),
        pl.BlockSpec((BLK, D), lambda i: (i, 0)),
    ],
    out_shape=[
        jax.ShapeDtypeStruct((N, D), jnp.float32),
        jax.ShapeDtypeStruct((N, D), jnp.bfloat16),
    ],
)


def _tc_head_body(p_ref, hp_ref, dinv_ref, b3_ref,
                  wp1_ref, bp1_ref, wp2_ref, bp2_ref,
                  wv1_ref, bv1_ref, wv2_ref, bv2_ref,
                  lo_ref, vo_ref, acc_ref):
    i = pl.program_id(0)
    s = (p_ref[0].astype(jnp.float32) + p_ref[1].astype(jnp.float32)
         + hp_ref[...])
    t = s * dinv_ref[...] + b3_ref[...]
    csum = jnp.sum(t, axis=0, keepdims=True)

    @pl.when(i == 0)
    def _():
        acc_ref[...] = csum

    @pl.when(i > 0)
    def _():
        acc_ref[...] += csum

    @pl.when(i == GRID - 1)
    def _():
        g = acc_ref[...] * (1.0 / N)
        hp_pol = jnp.maximum(
            jnp.dot(g, wp1_ref[...], preferred_element_type=jnp.float32,
                   precision=lax.Precision.HIGHEST)
            + bp1_ref[...], 0.0)
        lo_ref[...] = jnp.dot(hp_pol, wp2_ref[...],
                              preferred_element_type=jnp.float32,
                   precision=lax.Precision.HIGHEST) + bp2_ref[...]
        hp_val = jnp.maximum(
            jnp.dot(g, wv1_ref[...], preferred_element_type=jnp.float32,
                   precision=lax.Precision.HIGHEST)
            + bv1_ref[...], 0.0)
        vo_ref[...] = jnp.dot(hp_val, wv2_ref[...],
                              preferred_element_type=jnp.float32,
                   precision=lax.Precision.HIGHEST) + bv2_ref[...]


_tc_head = pl.pallas_call(
    _tc_head_body,
    grid=(GRID,),
    in_specs=[
        pl.BlockSpec((NC, BLK, D), lambda i: (0, i, 0)),
        pl.BlockSpec((BLK, D), lambda i: (i, 0)),
        pl.BlockSpec((BLK, 1), lambda i: (i, 0)),
        pl.BlockSpec((1, D), lambda i: (0, 0)),
        pl.BlockSpec((D, D), lambda i: (0, 0)),
        pl.BlockSpec((1, D), lambda i: (0, 0)),
        pl.BlockSpec((D, D), lambda i: (0, 0)),
        pl.BlockSpec((1, D), lambda i: (0, 0)),
        pl.BlockSpec((D, D), lambda i: (0, 0)),
        pl.BlockSpec((1, D), lambda i: (0, 0)),
        pl.BlockSpec((D, D), lambda i: (0, 0)),
        pl.BlockSpec((1, D), lambda i: (0, 0)),
    ],
    out_specs=[
        pl.BlockSpec((1, D), lambda i: (0, 0)),
        pl.BlockSpec((1, D), lambda i: (0, 0)),
    ],
    out_shape=[
        jax.ShapeDtypeStruct((1, D), jnp.float32),
        jax.ShapeDtypeStruct((1, D), jnp.float32),
    ],
    scratch_shapes=[pltpu.VMEM((1, D), jnp.float32)],
)


def kernel(x, edge_index, W1, b1, W2, b2, W3, b3,
           Wp1, bp1, Wp2, bp2, Wv1, bv1, Wv2, bv2):
    src = edge_index[0]
    dst = edge_index[1]
    pad = E_PAD - E
    srcp = jnp.concatenate(
        [src, jnp.zeros((pad,), jnp.int32)]).reshape(NW, CHUNKS, CH)
    dstp = jnp.concatenate(
        [dst, jnp.full((pad,), DUMP, jnp.int32)]).reshape(NW, CHUNKS, CH)
    zz = jnp.zeros((RPT, D), jnp.bfloat16)
    zz16 = jnp.zeros((RPT, DEG_W), jnp.float32)
    ones16 = jnp.ones((CH, DEG_W), jnp.float32)

    degp = _sc_degree(dstp, ones16, zz16)
    hp1, hb1, dinv = _tc_input(x, W1, degp, degp)
    p1 = _sc_propagate(hb1, srcp, dstp, zz)
    hp2, hb2 = _tc_mid(p1, hp1, dinv, b1.reshape(1, D), W2)
    p2 = _sc_propagate(hb2, srcp, dstp, zz)
    hp3, hb3 = _tc_mid(p2, hp2, dinv, b2.reshape(1, D), W3)
    p3 = _sc_propagate(hb3, srcp, dstp, zz)

    wp2p = jnp.zeros((D, D), jnp.float32).at[:, :A].set(Wp2)
    bp2p = jnp.zeros((1, D), jnp.float32).at[0, :A].set(bp2)
    wv2p = jnp.zeros((D, D), jnp.float32).at[:, :1].set(Wv2)
    bv2p = jnp.zeros((1, D), jnp.float32).at[0, :1].set(bv2)
    logits, value = _tc_head(
        p3, hp3, dinv, b3.reshape(1, D),
        Wp1, bp1.reshape(1, D), wp2p, bp2p,
        Wv1, bv1.reshape(1, D), wv2p, bv2p)
    return logits[:, :A], value[:, :1]


# speedup vs baseline: 14.6150x; 1.2346x over previous
import functools

import jax
import jax.numpy as jnp
from jax import lax
from jax.experimental import pallas as pl
from jax.experimental.pallas import tpu as pltpu
from jax.experimental.pallas import tpu_sc as plsc

N = 10000
D = 128
A = 6
E = 320000
NC, NS = 2, 16
NW = NC * NS
CH = 128
CHUNKS = 80
EPW = CH * CHUNKS
E_PAD = EPW * NW
DUMP = N
N_ACC = N + 112
RPT = N_ACC // NS
DEG_W = 16
BLK = 400
GRID = N // BLK

_MESH = plsc.VectorSubcoreMesh(core_axis_name="c", subcore_axis_name="s")


@functools.partial(
    pl.kernel,
    out_type=jax.ShapeDtypeStruct((NC, N_ACC, DEG_W), jnp.float32),
    mesh=_MESH,
    scratch_types=[
        pltpu.VMEM((CHUNKS, CH), jnp.int32),
        pltpu.VMEM((CH, DEG_W), jnp.float32),
        pltpu.VMEM_SHARED((N_ACC, DEG_W), jnp.float32),
    ],
    compiler_params=pltpu.CompilerParams(use_tc_tiling_on_sc=False),
)
def _sc_degree(dst_hbm, ones_hbm, zz_hbm, out_hbm, dst_v, ones_v, acc_sh):
    c = lax.axis_index("c")
    s = lax.axis_index("s")
    wid = s * NC + c
    pltpu.sync_copy(dst_hbm.at[wid], dst_v)
    pltpu.sync_copy(ones_hbm, ones_v)
    pltpu.sync_copy(zz_hbm, acc_sh.at[pl.ds(s * RPT, RPT)])
    plsc.subcore_barrier()

    def step(j, carry):
        pltpu.sync_copy(ones_v, acc_sh.at[dst_v.at[j]], add=True)
        return carry

    lax.fori_loop(0, CHUNKS, step, 0, unroll=False)
    plsc.subcore_barrier()
    pltpu.sync_copy(acc_sh.at[pl.ds(s * RPT, RPT)],
                    out_hbm.at[c, pl.ds(s * RPT, RPT)])


NBUF = 4
PRE = 2


@functools.partial(
    pl.kernel,
    out_type=jax.ShapeDtypeStruct((NC, N_ACC, D), jnp.bfloat16),
    mesh=_MESH,
    scratch_types=[
        pltpu.VMEM((CHUNKS, CH), jnp.int32),
        pltpu.VMEM((CHUNKS, CH), jnp.int32),
        pltpu.VMEM((NBUF, CH, D), jnp.bfloat16),
    ] + [pltpu.SemaphoreType.DMA] * (2 * NBUF) + [
        pltpu.VMEM_SHARED((N_ACC, D), jnp.bfloat16),
    ],
    compiler_params=pltpu.CompilerParams(use_tc_tiling_on_sc=False),
)
def _sc_propagate(hp_hbm, src_hbm, dst_hbm, zz_hbm, out_hbm,
                  src_v, dst_v, rows_v,
                  g0, g1, g2, g3, s0, s1, s2, s3, acc_sh):
    gsems = (g0, g1, g2, g3)
    ssems = (s0, s1, s2, s3)
    c = lax.axis_index("c")
    s = lax.axis_index("s")
    wid = s * NC + c
    pltpu.sync_copy(src_hbm.at[wid], src_v)
    pltpu.sync_copy(dst_hbm.at[wid], dst_v)
    pltpu.sync_copy(zz_hbm, acc_sh.at[pl.ds(s * RPT, RPT)])
    plsc.subcore_barrier()

    def gather(j, b):
        pltpu.async_copy(hp_hbm.at[src_v.at[j]], rows_v.at[b], gsems[b])

    def wait_gather(b):
        pltpu.make_async_copy(hp_hbm.at[src_v.at[0]],
                              rows_v.at[b], gsems[b]).wait()

    def scatter(j, b):
        pltpu.async_copy(rows_v.at[b], acc_sh.at[dst_v.at[j]], ssems[b],
                         add=True)

    def wait_scatter(b):
        pltpu.make_async_copy(rows_v.at[b], acc_sh.at[dst_v.at[0]],
                              ssems[b]).wait()

    for b in range(PRE):
        gather(b, b)

    def group(g, carry):
        for b in range(NBUF):
            j = g * NBUF + b
            wait_gather(b)
            scatter(j, b)
            bg = (b + PRE) % NBUF

            @pl.when(j >= NBUF - PRE)
            def _():
                wait_scatter(bg)

            @pl.when(j + PRE < CHUNKS)
            def _():
                gather(j + PRE, bg)
        return carry

    lax.fori_loop(0, CHUNKS // NBUF, group, 0, unroll=False)
    for j in range(CHUNKS - NBUF + PRE, CHUNKS):
        wait_scatter(j % NBUF)
    plsc.subcore_barrier()
    pltpu.sync_copy(acc_sh.at[pl.ds(s * RPT, RPT)],
                    out_hbm.at[c, pl.ds(s * RPT, RPT)])


def _tc_in_body(x_ref, w_ref, d0_ref, d1_ref, hp_ref, hb_ref, dinv_ref):
    deg = d0_ref[0, :, :1] + d1_ref[0, :, :1] + 1.0
    dinv = lax.rsqrt(deg)
    h = jnp.dot(x_ref[...], w_ref[...], preferred_element_type=jnp.float32,
                   precision=lax.Precision.HIGHEST)
    hp = h * dinv
    hp_ref[...] = hp
    hb_ref[...] = hp.astype(jnp.bfloat16)
    dinv_ref[...] = dinv


_tc_input = pl.pallas_call(
    _tc_in_body,
    grid=(GRID,),
    in_specs=[
        pl.BlockSpec((BLK, D), lambda i: (i, 0)),
        pl.BlockSpec((D, D), lambda i: (0, 0)),
        pl.BlockSpec((1, BLK, DEG_W), lambda i: (0, i, 0)),
        pl.BlockSpec((1, BLK, DEG_W), lambda i: (1, i, 0)),
    ],
    out_specs=[
        pl.BlockSpec((BLK, D), lambda i: (i, 0)),
        pl.BlockSpec((BLK, D), lambda i: (i, 0)),
        pl.BlockSpec((BLK, 1), lambda i: (i, 0)),
    ],
    out_shape=[
        jax.ShapeDtypeStruct((N, D), jnp.float32),
        jax.ShapeDtypeStruct((N, D), jnp.bfloat16),
        jax.ShapeDtypeStruct((N, 1), jnp.float32),
    ],
)


def _tc_mid_body(p_ref, hp_ref, dinv_ref, b_ref, w_ref, out_ref, ob_ref):
    s = (p_ref[0].astype(jnp.float32) + p_ref[1].astype(jnp.float32)
         + hp_ref[...])
    dinv = dinv_ref[...]
    t = jnp.maximum(s * dinv + b_ref[...], 0.0)
    o = jnp.dot(t, w_ref[...],
                preferred_element_type=jnp.float32,
                precision=lax.Precision.HIGHEST) * dinv
    out_ref[...] = o
    ob_ref[...] = o.astype(jnp.bfloat16)


_tc_mid = pl.pallas_call(
    _tc_mid_body,
    grid=(GRID,),
    in_specs=[
        pl.BlockSpec((NC, BLK, D), lambda i: (0, i, 0)),
        pl.BlockSpec((BLK, D), lambda i: (i, 0)),
        pl.BlockSpec((BLK, 1), lambda i: (i, 0)),
        pl.BlockSpec((1, D), lambda i: (0, 0)),
        pl.BlockSpec((D, D), lambda i: (0, 0)),
    ],
    out_specs=[
        pl.BlockSpec((BLK, D), lambda i: (i, 0)),
        pl.BlockSpec((BLK, D), lambda i: (i, 0)),
    ],
    out_shape=[
        jax.ShapeDtypeStruct((N, D), jnp.float32),
        jax.ShapeDtypeStruct((N, D), jnp.bfloat16),
    ],
)


def _tc_head_body(p_ref, hp_ref, dinv_ref, b3_ref,
                  wp1_ref, bp1_ref, wp2_ref, bp2_ref,
                  wv1_ref, bv1_ref, wv2_ref, bv2_ref,
                  lo_ref, vo_ref, acc_ref):
    i = pl.program_id(0)
    s = (p_ref[0].astype(jnp.float32) + p_ref[1].astype(jnp.float32)
         + hp_ref[...])
    t = s * dinv_ref[...] + b3_ref[...]
    csum = jnp.sum(t, axis=0, keepdims=True)

    @pl.when(i == 0)
    def _():
        acc_ref[...] = csum

    @pl.when(i > 0)
    def _():
        acc_ref[...] += csum

    @pl.when(i == GRID - 1)
    def _():
        g = acc_ref[...] * (1.0 / N)
        hp_pol = jnp.maximum(
            jnp.dot(g, wp1_ref[...], preferred_element_type=jnp.float32,
                   precision=lax.Precision.HIGHEST)
            + bp1_ref[...], 0.0)
        lo_ref[...] = jnp.dot(hp_pol, wp2_ref[...],
                              preferred_element_type=jnp.float32,
                   precision=lax.Precision.HIGHEST) + bp2_ref[...]
        hp_val = jnp.maximum(
            jnp.dot(g, wv1_ref[...], preferred_element_type=jnp.float32,
                   precision=lax.Precision.HIGHEST)
            + bv1_ref[...], 0.0)
        vo_ref[...] = jnp.dot(hp_val, wv2_ref[...],
                              preferred_element_type=jnp.float32,
                   precision=lax.Precision.HIGHEST) + bv2_ref[...]


_tc_head = pl.pallas_call(
    _tc_head_body,
    grid=(GRID,),
    in_specs=[
        pl.BlockSpec((NC, BLK, D), lambda i: (0, i, 0)),
        pl.BlockSpec((BLK, D), lambda i: (i, 0)),
        pl.BlockSpec((BLK, 1), lambda i: (i, 0)),
        pl.BlockSpec((1, D), lambda i: (0, 0)),
        pl.BlockSpec((D, D), lambda i: (0, 0)),
        pl.BlockSpec((1, D), lambda i: (0, 0)),
        pl.BlockSpec((D, D), lambda i: (0, 0)),
        pl.BlockSpec((1, D), lambda i: (0, 0)),
        pl.BlockSpec((D, D), lambda i: (0, 0)),
        pl.BlockSpec((1, D), lambda i: (0, 0)),
        pl.BlockSpec((D, D), lambda i: (0, 0)),
        pl.BlockSpec((1, D), lambda i: (0, 0)),
    ],
    out_specs=[
        pl.BlockSpec((1, D), lambda i: (0, 0)),
        pl.BlockSpec((1, D), lambda i: (0, 0)),
    ],
    out_shape=[
        jax.ShapeDtypeStruct((1, D), jnp.float32),
        jax.ShapeDtypeStruct((1, D), jnp.float32),
    ],
    scratch_shapes=[pltpu.VMEM((1, D), jnp.float32)],
)


def kernel(x, edge_index, W1, b1, W2, b2, W3, b3,
           Wp1, bp1, Wp2, bp2, Wv1, bv1, Wv2, bv2):
    src = edge_index[0]
    dst = edge_index[1]
    pad = E_PAD - E
    srcp = jnp.concatenate(
        [src, jnp.zeros((pad,), jnp.int32)]
    ).reshape(CH, NW, CHUNKS).transpose(1, 2, 0)
    dstp = jnp.concatenate(
        [dst, jnp.full((pad,), DUMP, jnp.int32)]
    ).reshape(CH, NW, CHUNKS).transpose(1, 2, 0)
    zz = jnp.zeros((RPT, D), jnp.bfloat16)
    zz16 = jnp.zeros((RPT, DEG_W), jnp.float32)
    ones16 = jnp.ones((CH, DEG_W), jnp.float32)

    degp = _sc_degree(dstp, ones16, zz16)
    hp1, hb1, dinv = _tc_input(x, W1, degp, degp)
    p1 = _sc_propagate(hb1, srcp, dstp, zz)
    hp2, hb2 = _tc_mid(p1, hp1, dinv, b1.reshape(1, D), W2)
    p2 = _sc_propagate(hb2, srcp, dstp, zz)
    hp3, hb3 = _tc_mid(p2, hp2, dinv, b2.reshape(1, D), W3)
    p3 = _sc_propagate(hb3, srcp, dstp, zz)

    wp2p = jnp.zeros((D, D), jnp.float32).at[:, :A].set(Wp2)
    bp2p = jnp.zeros((1, D), jnp.float32).at[0, :A].set(bp2)
    wv2p = jnp.zeros((D, D), jnp.float32).at[:, :1].set(Wv2)
    bv2p = jnp.zeros((1, D), jnp.float32).at[0, :1].set(bv2)
    logits, value = _tc_head(
        p3, hp3, dinv, b3.reshape(1, D),
        Wp1, bp1.reshape(1, D), wp2p, bp2p,
        Wv1, bv1.reshape(1, D), wv2p, bv2p)
    return logits[:, :A], value[:, :1]
